# Initial kernel scaffold; baseline (speedup 1.0000x reference)
import jax
import jax.numpy as jnp
from jax.experimental import pallas as pl


def kernel(x, charges):
    raise NotImplementedError("write your pallas kernel here")


# speedup vs baseline: 1.2040x; 1.2040x over previous
import jax
import jax.numpy as jnp
from jax.experimental import pallas as pl

_N = 2048
_BOX = 24.0
_HALF = 12.0
_CUTOFF_SQ = 2.5 * 2.5

_K_E = 8987551787.0
_EV_TO_J = 1.6021e-19
_ASE_C = 6.241509074460763e+18
_ASE_M = 1.0e10
_CONVERSION = _K_E * (_ASE_C ** -2) * (1.0 / _EV_TO_J) * _ASE_M

_BI = 256


def _coulomb_body(xi_ref, xjt_ref, q_ref, o_ref):
    gi = pl.program_id(0)
    i0 = gi * _BI

    d2 = jnp.zeros((_BI, _N), jnp.float32)
    for k in range(3):
        xi = xi_ref[:, k:k + 1]
        xj = xjt_ref[k:k + 1, :]
        d = xj - xi
        off = (jnp.where(d < -_HALF, _BOX, 0.0)
               - jnp.where(d >= _HALF, _BOX, 0.0))
        d = d + off
        d2 = d2 + d * d

    ii = jax.lax.broadcasted_iota(jnp.int32, (_BI, _N), 0) + i0
    jj = jax.lax.broadcasted_iota(jnp.int32, (_BI, _N), 1)
    mask = (d2 < _CUTOFF_SQ) & (d2 != 0.0) & (jj > ii)

    rinv = jax.lax.rsqrt(jnp.where(mask, d2, 1.0))
    q = q_ref[0:1, :]
    e = jnp.where(mask, (-_CONVERSION) * (q * q) * rinv, 0.0)
    s = jnp.sum(e).reshape(1, 1)

    @pl.when(gi == 0)
    def _init():
        o_ref[...] = s

    @pl.when(gi != 0)
    def _accum():
        o_ref[...] += s


def kernel(x, charges):
    xjt = x.T
    q = charges.reshape(1, _N)
    out = pl.pallas_call(
        _coulomb_body,
        grid=(_N // _BI,),
        in_specs=[
            pl.BlockSpec((_BI, 3), lambda i: (i, 0)),
            pl.BlockSpec((3, _N), lambda i: (0, 0)),
            pl.BlockSpec((1, _N), lambda i: (0, 0)),
        ],
        out_specs=pl.BlockSpec((1, 1), lambda i: (0, 0)),
        out_shape=jax.ShapeDtypeStruct((1, 1), jnp.float32),
    )(x, xjt, q)
    return out[0, 0]


# speedup vs baseline: 2.1735x; 1.8052x over previous
import jax
import jax.numpy as jnp
from jax.experimental import pallas as pl

_N = 2048
_BOX = 24.0
_INV_BOX = 1.0 / 24.0
_CUTOFF_SQ = 2.5 * 2.5

_K_E = 8987551787.0
_EV_TO_J = 1.6021e-19
_ASE_C = 6.241509074460763e+18
_ASE_M = 1.0e10
_CONVERSION = _K_E * (_ASE_C ** -2) * (1.0 / _EV_TO_J) * _ASE_M

_BI = 256
_BJ = 256
_NBLK = _N // _BI


def _coulomb_body(xi_ref, xjt_ref, q_ref, o_ref):
    gi = pl.program_id(0)
    i0 = gi * _BI

    ii = jax.lax.broadcasted_iota(jnp.int32, (_BI, _BJ), 0) + i0
    jj_loc = jax.lax.broadcasted_iota(jnp.int32, (_BI, _BJ), 1)

    def chunk(c, acc):
        j0 = c * _BJ
        xj = xjt_ref[:, pl.ds(j0, _BJ)]
        d2 = jnp.zeros((_BI, _BJ), jnp.float32)
        for k in range(3):
            d = xj[k:k + 1, :] - xi_ref[:, k:k + 1]
            d = d - _BOX * jnp.round(d * _INV_BOX)
            d2 = d2 + d * d
        m = (d2 < _CUTOFF_SQ) & (d2 != 0.0) & (jj_loc + j0 > ii)
        rinv = jax.lax.rsqrt(jnp.where(m, d2, 1.0))
        qc = q_ref[0:1, pl.ds(j0, _BJ)]
        e = jnp.where(m, ((-_CONVERSION) * qc * qc) * rinv, 0.0)
        return acc + jnp.sum(e)

    s = jax.lax.fori_loop(gi, _NBLK, chunk, 0.0).reshape(1, 1)

    @pl.when(gi == 0)
    def _init():
        o_ref[...] = s

    @pl.when(gi != 0)
    def _accum():
        o_ref[...] += s


def kernel(x, charges):
    xjt = x.T
    q = charges.reshape(1, _N)
    out = pl.pallas_call(
        _coulomb_body,
        grid=(_NBLK,),
        in_specs=[
            pl.BlockSpec((_BI, 3), lambda i: (i, 0)),
            pl.BlockSpec((3, _N), lambda i: (0, 0)),
            pl.BlockSpec((1, _N), lambda i: (0, 0)),
        ],
        out_specs=pl.BlockSpec((1, 1), lambda i: (0, 0)),
        out_shape=jax.ShapeDtypeStruct((1, 1), jnp.float32),
    )(x, xjt, q)
    return out[0, 0]


# speedup vs baseline: 2.3090x; 1.0623x over previous
import jax
import jax.numpy as jnp
from jax.experimental import pallas as pl

_N = 2048
_BOX = 24.0
_INV_BOX = 1.0 / 24.0
_CUTOFF_SQ = 2.5 * 2.5

_K_E = 8987551787.0
_EV_TO_J = 1.6021e-19
_ASE_C = 6.241509074460763e+18
_ASE_M = 1.0e10
_CONVERSION = _K_E * (_ASE_C ** -2) * (1.0 / _EV_TO_J) * _ASE_M

_BI = 256
_BJ = 256
_NBLK = _N // _BI
_INF = float("inf")


def _coulomb_body(xi_ref, xjt_ref, q_ref, o_ref):
    gi = pl.program_id(0)

    def chunk_energy(j0, extra_mask):
        xj = xjt_ref[:, pl.ds(j0, _BJ)]
        d2 = jnp.zeros((_BI, _BJ), jnp.float32)
        for k in range(3):
            d = xj[k:k + 1, :] - xi_ref[:, k:k + 1]
            d = d - _BOX * jnp.round(d * _INV_BOX)
            d2 = d2 + d * d
        m = (d2 < _CUTOFF_SQ) & (d2 != 0.0)
        if extra_mask is not None:
            m = m & extra_mask
        qc = q_ref[0:1, pl.ds(j0, _BJ)]
        rinv = jax.lax.rsqrt(jnp.where(m, d2, _INF))
        return jnp.sum(((-_CONVERSION) * qc * qc) * rinv)

    tri = (jax.lax.broadcasted_iota(jnp.int32, (_BI, _BJ), 1)
           > jax.lax.broadcasted_iota(jnp.int32, (_BI, _BJ), 0))
    s0 = chunk_energy(gi * _BI, tri)

    def chunk(c, acc):
        return acc + chunk_energy(c * _BJ, None)

    s = jax.lax.fori_loop(gi + 1, _NBLK, chunk, s0).reshape(1, 1)

    @pl.when(gi == 0)
    def _init():
        o_ref[...] = s

    @pl.when(gi != 0)
    def _accum():
        o_ref[...] += s


def kernel(x, charges):
    xjt = x.T
    q = charges.reshape(1, _N)
    out = pl.pallas_call(
        _coulomb_body,
        grid=(_NBLK,),
        in_specs=[
            pl.BlockSpec((_BI, 3), lambda i: (i, 0)),
            pl.BlockSpec((3, _N), lambda i: (0, 0)),
            pl.BlockSpec((1, _N), lambda i: (0, 0)),
        ],
        out_specs=pl.BlockSpec((1, 1), lambda i: (0, 0)),
        out_shape=jax.ShapeDtypeStruct((1, 1), jnp.float32),
    )(x, xjt, q)
    return out[0, 0]


# speedup vs baseline: 2.9504x; 1.2778x over previous
import jax
import jax.numpy as jnp
from jax.experimental import pallas as pl

_N = 2048
_H = 1024
_BOX = 24.0
_CUTOFF_SQ = 2.5 * 2.5

_K_E = 8987551787.0
_EV_TO_J = 1.6021e-19
_ASE_C = 6.241509074460763e+18
_ASE_M = 1.0e10
_CONVERSION = _K_E * (_ASE_C ** -2) * (1.0 / _EV_TO_J) * _ASE_M

_INF = float("inf")


def _coulomb_body(xlo_ref, xhi_ref, xlot_ref, xhit_ref, qlo_ref, qhi_ref,
                  qhic_ref, o_ref):
    qlo = qlo_ref[0:1, :]
    qhi = qhi_ref[0:1, :]
    wlo_row = (-_CONVERSION) * qlo * qlo
    whi_row = (-_CONVERSION) * qhi * qhi
    qhic = qhic_ref[:, 0:1]
    whi_col = (-_CONVERSION) * qhic * qhic

    cgtr = (jax.lax.broadcasted_iota(jnp.int32, (_H, _H), 1)
            > jax.lax.broadcasted_iota(jnp.int32, (_H, _H), 0))

    d2 = jnp.zeros((_H, _H), jnp.float32)
    for k in range(3):
        xr = jnp.where(cgtr, xlo_ref[:, k:k + 1], xhi_ref[:, k:k + 1])
        xc = jnp.where(cgtr, xlot_ref[k:k + 1, :], xhit_ref[k:k + 1, :])
        a = jnp.abs(xc - xr)
        w = jnp.minimum(a, _BOX - a)
        d2 = d2 + w * w
    m = (d2 < _CUTOFF_SQ) & (d2 != 0.0)
    wt = jnp.where(cgtr, wlo_row, whi_col)
    s = jnp.sum(wt * jax.lax.rsqrt(jnp.where(m, d2, _INF)))

    d2 = jnp.zeros((_H, _H), jnp.float32)
    for k in range(3):
        a = jnp.abs(xhit_ref[k:k + 1, :] - xlo_ref[:, k:k + 1])
        w = jnp.minimum(a, _BOX - a)
        d2 = d2 + w * w
    m = (d2 < _CUTOFF_SQ) & (d2 != 0.0)
    s = s + jnp.sum(whi_row * jax.lax.rsqrt(jnp.where(m, d2, _INF)))

    o_ref[...] = s.reshape(1, 1)


def kernel(x, charges):
    xlo = x[:_H]
    xhi = x[_H:]
    xlot = xlo.T
    xhit = xhi.T
    qlo = charges[:_H].reshape(1, _H)
    qhi = charges[_H:].reshape(1, _H)
    qhic = charges[_H:].reshape(_H, 1)
    out = pl.pallas_call(
        _coulomb_body,
        out_shape=jax.ShapeDtypeStruct((1, 1), jnp.float32),
    )(xlo, xhi, xlot, xhit, qlo, qhi, qhic)
    return out[0, 0]
